# baseline (speedup 1.0000x reference)
import functools

import jax
import jax.numpy as jnp
from jax import lax
from jax.experimental import pallas as pl
from jax.experimental.pallas import tpu as pltpu
from jax.experimental.pallas import tpu_sc as plsc

_NT = 6000
_BN = 600
_BNC = 400
_CH = 8
_NC = 2
_NS = 16
_NW = _NC * _NS


def _head_body(prev_ref, neigh_ref, w_ref, out_ref):
    s = jnp.sum(neigh_ref[...], axis=1) + prev_ref[...]
    acc = jnp.dot(s, w_ref[...], preferred_element_type=jnp.float32)
    out_ref[...] = jnp.maximum(acc, 0.0)


def _tail_body(_, means_ref, w_ref, out_ref):
    acc = jnp.dot(means_ref[...], w_ref[...], preferred_element_type=jnp.float32)
    out_ref[...] = jnp.maximum(acc, 0.0)


def _sc_means(prev_hidden, neigh_hidden, nt):
    n, d = prev_hidden.shape
    k = neigh_hidden.shape[1]
    nr = n - nt
    nchunk_total = nr // _CH
    base_q, rem = divmod(nchunk_total, _NW)
    mesh = plsc.VectorSubcoreMesh(core_axis_name="c", subcore_axis_name="s")

    cpw = base_q + (1 if rem else 0)

    @functools.partial(
        pl.kernel,
        out_type=jax.ShapeDtypeStruct((nr, d), jnp.float32),
        mesh=mesh,
        scratch_types=[
            pltpu.VMEM((2, _CH, k, d), jnp.float32),
            pltpu.VMEM((2, _CH, d), jnp.float32),
            pltpu.VMEM((_CH, d), jnp.float32),
            pltpu.SemaphoreType.DMA,
            pltpu.SemaphoreType.DMA,
        ],
    )
    def sc_kernel(prev_hbm, neigh_hbm, means_hbm, neigh_v, prev_v, out_v,
                  sem0, sem1):
        wid = lax.axis_index("s") * _NC + lax.axis_index("c")
        c_start = wid * base_q + jnp.minimum(wid, rem)
        c_last = c_start + base_q + jnp.where(wid < rem, 1, 0) - 1
        sems = (sem0, sem1)
        ngrp = d // 16

        def start_fetch(i):
            ci = jnp.minimum(c_start + i, c_last)
            b = i % 2
            r0 = ci * _CH
            dn = pltpu.async_copy(
                neigh_hbm.at[pl.ds(nt + r0, _CH)], neigh_v.at[b], sems[b])
            dp = pltpu.async_copy(
                prev_hbm.at[pl.ds(nt + r0, _CH)], prev_v.at[b], sems[b])
            return (dn, dp)

        pending = start_fetch(0)
        for i in range(cpw):
            b = i % 2
            nxt = start_fetch(i + 1) if i + 1 < cpw else None
            pending[0].wait()
            pending[1].wait()
            pending = nxt

            @plsc.parallel_loop(0, _CH * ngrp, unroll=2)
            def _(idx):
                sh = ngrp.bit_length() - 1
                r = lax.shift_right_logical(idx, sh)
                col = pl.multiple_of(
                    lax.shift_left(jnp.bitwise_and(idx, ngrp - 1), 4), 16)
                acc = prev_v[b, r, pl.ds(col, 16)]
                for kk in range(k):
                    acc = acc + neigh_v[b, r, kk, pl.ds(col, 16)]
                out_v[r, pl.ds(col, 16)] = acc

            ci = jnp.minimum(c_start + i, c_last)
            pltpu.sync_copy(out_v, means_hbm.at[pl.ds(ci * _CH, _CH)])

    return sc_kernel(prev_hidden, neigh_hidden)


def kernel(prev_hidden, neigh_hidden, W):
    n, d = prev_hidden.shape
    k = neigh_hidden.shape[1]
    f = W.shape[1]
    nt = _NT
    nr = n - nt

    w_scaled = W * (1.0 / (k + 1))

    sums = _sc_means(prev_hidden, neigh_hidden, nt)

    out_head = pl.pallas_call(
        _head_body,
        grid=(nt // _BN,),
        in_specs=[
            pl.BlockSpec((_BN, d), lambda i: (i, 0)),
            pl.BlockSpec((_BN, k, d), lambda i: (i, 0, 0)),
            pl.BlockSpec((d, f), lambda i: (0, 0)),
        ],
        out_specs=pl.BlockSpec((_BN, f), lambda i: (i, 0)),
        out_shape=jax.ShapeDtypeStruct((n, f), jnp.float32),
    )(prev_hidden, neigh_hidden, w_scaled)

    off = nt // _BNC
    out = pl.pallas_call(
        _tail_body,
        grid=(nr // _BNC,),
        in_specs=[
            pl.BlockSpec(memory_space=pltpu.MemorySpace.HBM),
            pl.BlockSpec((_BNC, d), lambda i: (i, 0)),
            pl.BlockSpec((d, f), lambda i: (0, 0)),
        ],
        out_specs=pl.BlockSpec((_BNC, f), lambda i: (i + off, 0)),
        out_shape=jax.ShapeDtypeStruct((n, f), jnp.float32),
        input_output_aliases={0: 0},
    )(out_head, sums, w_scaled)
    return out


# speedup vs baseline: 1.0099x; 1.0099x over previous
import functools

import jax
import jax.numpy as jnp
from jax import lax
from jax.experimental import pallas as pl
from jax.experimental.pallas import tpu as pltpu
from jax.experimental.pallas import tpu_sc as plsc

_NT = 6000
_BN = 1000
_BNC = 1000
_CH = 8
_NC = 2
_NS = 16
_NW = _NC * _NS


def _head_body(prev_ref, neigh_a_ref, neigh_b_ref, w_ref, out_ref):
    k = neigh_a_ref.shape[1] + neigh_b_ref.shape[1]
    s = (jnp.sum(neigh_a_ref[...], axis=1) + jnp.sum(neigh_b_ref[...], axis=1)
         + prev_ref[...])
    means = s * (1.0 / (k + 1))
    acc = jnp.dot(means, w_ref[...], preferred_element_type=jnp.float32)
    out_ref[...] = jnp.maximum(acc, 0.0)


def _tail_body(_, means_ref, w_ref, out_ref):
    acc = jnp.dot(means_ref[...], w_ref[...], preferred_element_type=jnp.float32)
    out_ref[...] = jnp.maximum(acc, 0.0)


def _sc_means(prev_hidden, neigh_hidden, nt):
    n, d = prev_hidden.shape
    k = neigh_hidden.shape[1]
    nr = n - nt
    nchunk_total = nr // _CH
    base_q, rem = divmod(nchunk_total, _NW)
    inv = 1.0 / (k + 1)
    mesh = plsc.VectorSubcoreMesh(core_axis_name="c", subcore_axis_name="s")

    cpw = base_q + (1 if rem else 0)

    @functools.partial(
        pl.kernel,
        out_type=jax.ShapeDtypeStruct((nr, d), jnp.float32),
        mesh=mesh,
        scratch_types=[
            pltpu.VMEM((2, _CH, k, d), jnp.float32),
            pltpu.VMEM((2, _CH, d), jnp.float32),
            pltpu.VMEM((_CH, d), jnp.float32),
            pltpu.SemaphoreType.DMA,
            pltpu.SemaphoreType.DMA,
        ],
    )
    def sc_kernel(prev_hbm, neigh_hbm, means_hbm, neigh_v, prev_v, out_v,
                  sem0, sem1):
        wid = lax.axis_index("s") * _NC + lax.axis_index("c")
        c_start = wid * base_q + jnp.minimum(wid, rem)
        c_last = c_start + base_q + jnp.where(wid < rem, 1, 0) - 1
        sems = (sem0, sem1)
        ngrp = d // 16

        def start_fetch(i):
            ci = jnp.minimum(c_start + i, c_last)
            b = i % 2
            r0 = ci * _CH
            dn = pltpu.async_copy(
                neigh_hbm.at[pl.ds(nt + r0, _CH)], neigh_v.at[b], sems[b])
            dp = pltpu.async_copy(
                prev_hbm.at[pl.ds(nt + r0, _CH)], prev_v.at[b], sems[b])
            return (dn, dp)

        pending = start_fetch(0)
        for i in range(cpw):
            b = i % 2
            nxt = start_fetch(i + 1) if i + 1 < cpw else None
            pending[0].wait()
            pending[1].wait()
            pending = nxt

            @plsc.parallel_loop(0, _CH * ngrp, unroll=2)
            def _(idx):
                sh = ngrp.bit_length() - 1
                r = lax.shift_right_logical(idx, sh)
                col = pl.multiple_of(
                    lax.shift_left(jnp.bitwise_and(idx, ngrp - 1), 4), 16)
                acc = prev_v[b, r, pl.ds(col, 16)]
                for kk in range(k):
                    acc = acc + neigh_v[b, r, kk, pl.ds(col, 16)]
                out_v[r, pl.ds(col, 16)] = acc * inv

            ci = jnp.minimum(c_start + i, c_last)
            pltpu.sync_copy(out_v, means_hbm.at[pl.ds(ci * _CH, _CH)])

    return sc_kernel(prev_hidden, neigh_hidden)


def kernel(prev_hidden, neigh_hidden, W):
    n, d = prev_hidden.shape
    k = neigh_hidden.shape[1]
    f = W.shape[1]
    nt = _NT
    nr = n - nt

    means = _sc_means(prev_hidden, neigh_hidden, nt)

    out_head = pl.pallas_call(
        _head_body,
        grid=(nt // _BN,),
        in_specs=[
            pl.BlockSpec((_BN, d), lambda i: (i, 0)),
            pl.BlockSpec((_BN, k // 2, d), lambda i: (i, 0, 0)),
            pl.BlockSpec((_BN, k // 2, d), lambda i: (i, 1, 0)),
            pl.BlockSpec((d, f), lambda i: (0, 0)),
        ],
        out_specs=pl.BlockSpec((_BN, f), lambda i: (i, 0)),
        out_shape=jax.ShapeDtypeStruct((n, f), jnp.float32),
    )(prev_hidden, neigh_hidden, neigh_hidden, W)

    off = nt // _BNC
    out = pl.pallas_call(
        _tail_body,
        grid=(nr // _BNC,),
        in_specs=[
            pl.BlockSpec(memory_space=pltpu.MemorySpace.HBM),
            pl.BlockSpec((_BNC, d), lambda i: (i, 0)),
            pl.BlockSpec((d, f), lambda i: (0, 0)),
        ],
        out_specs=pl.BlockSpec((_BNC, f), lambda i: (i + off, 0)),
        out_shape=jax.ShapeDtypeStruct((n, f), jnp.float32),
        input_output_aliases={0: 0},
    )(out_head, means, W)
    return out


# speedup vs baseline: 1.0301x; 1.0200x over previous
import functools

import jax
import jax.numpy as jnp
from jax import lax
from jax.experimental import pallas as pl
from jax.experimental.pallas import tpu as pltpu
from jax.experimental.pallas import tpu_sc as plsc

_NT = 6000
_BN = 1000
_BNC = 1000
_CH = 8
_NC = 2
_NS = 16
_NW = _NC * _NS


_NSUB = 8
_RS = _BN // _NSUB


def _head_body(neigh_hbm, prev_ref, w_ref, out_ref, nbuf, sems):
    ngrid = _NT // _BN
    i = pl.program_id(0)
    k = neigh_hbm.shape[1]
    d = neigh_hbm.shape[2]

    def copies(blk, slot):
        return [
            pltpu.make_async_copy(
                neigh_hbm.at[pl.ds(blk * _BN + s * _RS, _RS)],
                nbuf.at[slot, s],
                sems.at[slot, s],
            )
            for s in range(_NSUB)
        ]

    @pl.when(i == 0)
    def _():
        for c in copies(0, 0):
            c.start()

    @pl.when(i + 1 < ngrid)
    def _():
        for c in copies(i + 1, (i + 1) % 2):
            c.start()

    for c in copies(i, i % 2):
        c.wait()

    blk = nbuf[i % 2]
    s = jnp.sum(blk, axis=2).reshape(_BN, d) + prev_ref[...]
    means = s * (1.0 / (k + 1))
    acc = jnp.dot(means, w_ref[...], preferred_element_type=jnp.float32)
    out_ref[...] = jnp.maximum(acc, 0.0)


def _tail_body(_, means_ref, w_ref, out_ref):
    acc = jnp.dot(means_ref[...], w_ref[...], preferred_element_type=jnp.float32)
    out_ref[...] = jnp.maximum(acc, 0.0)


def _sc_means(prev_hidden, neigh_hidden, nt):
    n, d = prev_hidden.shape
    k = neigh_hidden.shape[1]
    nr = n - nt
    nchunk_total = nr // _CH
    base_q, rem = divmod(nchunk_total, _NW)
    inv = 1.0 / (k + 1)
    mesh = plsc.VectorSubcoreMesh(core_axis_name="c", subcore_axis_name="s")

    cpw = base_q + (1 if rem else 0)

    @functools.partial(
        pl.kernel,
        out_type=jax.ShapeDtypeStruct((nr, d), jnp.float32),
        mesh=mesh,
        scratch_types=[
            pltpu.VMEM((2, _CH, k, d), jnp.float32),
            pltpu.VMEM((2, _CH, d), jnp.float32),
            pltpu.VMEM((_CH, d), jnp.float32),
            pltpu.SemaphoreType.DMA,
            pltpu.SemaphoreType.DMA,
        ],
    )
    def sc_kernel(prev_hbm, neigh_hbm, means_hbm, neigh_v, prev_v, out_v,
                  sem0, sem1):
        wid = lax.axis_index("s") * _NC + lax.axis_index("c")
        c_start = wid * base_q + jnp.minimum(wid, rem)
        c_last = c_start + base_q + jnp.where(wid < rem, 1, 0) - 1
        sems = (sem0, sem1)
        ngrp = d // 16

        def start_fetch(i):
            ci = jnp.minimum(c_start + i, c_last)
            b = i % 2
            r0 = ci * _CH
            dn = pltpu.async_copy(
                neigh_hbm.at[pl.ds(nt + r0, _CH)], neigh_v.at[b], sems[b])
            dp = pltpu.async_copy(
                prev_hbm.at[pl.ds(nt + r0, _CH)], prev_v.at[b], sems[b])
            return (dn, dp)

        pending = start_fetch(0)
        for i in range(cpw):
            b = i % 2
            nxt = start_fetch(i + 1) if i + 1 < cpw else None
            pending[0].wait()
            pending[1].wait()
            pending = nxt

            @plsc.parallel_loop(0, _CH * ngrp, unroll=2)
            def _(idx):
                sh = ngrp.bit_length() - 1
                r = lax.shift_right_logical(idx, sh)
                col = pl.multiple_of(
                    lax.shift_left(jnp.bitwise_and(idx, ngrp - 1), 4), 16)
                acc = prev_v[b, r, pl.ds(col, 16)]
                for kk in range(k):
                    acc = acc + neigh_v[b, r, kk, pl.ds(col, 16)]
                out_v[r, pl.ds(col, 16)] = acc * inv

            ci = jnp.minimum(c_start + i, c_last)
            pltpu.sync_copy(out_v, means_hbm.at[pl.ds(ci * _CH, _CH)])

    return sc_kernel(prev_hidden, neigh_hidden)


def kernel(prev_hidden, neigh_hidden, W):
    n, d = prev_hidden.shape
    k = neigh_hidden.shape[1]
    f = W.shape[1]
    nt = _NT
    nr = n - nt

    means = _sc_means(prev_hidden, neigh_hidden, nt)

    out_head = pl.pallas_call(
        _head_body,
        grid=(nt // _BN,),
        in_specs=[
            pl.BlockSpec(memory_space=pltpu.MemorySpace.HBM),
            pl.BlockSpec((_BN, d), lambda i: (i, 0)),
            pl.BlockSpec((d, f), lambda i: (0, 0)),
        ],
        out_specs=pl.BlockSpec((_BN, f), lambda i: (i, 0)),
        out_shape=jax.ShapeDtypeStruct((n, f), jnp.float32),
        scratch_shapes=[
            pltpu.VMEM((2, _NSUB, _RS, k, d), jnp.float32),
            pltpu.SemaphoreType.DMA((2, _NSUB)),
        ],
    )(neigh_hidden, prev_hidden, W)

    off = nt // _BNC
    out = pl.pallas_call(
        _tail_body,
        grid=(nr // _BNC,),
        in_specs=[
            pl.BlockSpec(memory_space=pltpu.MemorySpace.HBM),
            pl.BlockSpec((_BNC, d), lambda i: (i, 0)),
            pl.BlockSpec((d, f), lambda i: (0, 0)),
        ],
        out_specs=pl.BlockSpec((_BNC, f), lambda i: (i + off, 0)),
        out_shape=jax.ShapeDtypeStruct((n, f), jnp.float32),
        input_output_aliases={0: 0},
    )(out_head, means, W)
    return out


# speedup vs baseline: 1.4320x; 1.3901x over previous
import functools

import jax
import jax.numpy as jnp
from jax import lax
from jax.experimental import pallas as pl
from jax.experimental.pallas import tpu as pltpu
from jax.experimental.pallas import tpu_sc as plsc

_NT = 10000
_BN = 1000
_BNC = 1000
_CH = 8
_NC = 2
_NS = 16
_NW = _NC * _NS


_NSUB = 8
_RS = _BN // _NSUB


def _head_body(neigh_hbm, prev_ref, w_ref, out_ref, nbuf, sems):
    ngrid = _NT // _BN
    i = pl.program_id(0)
    k = neigh_hbm.shape[1]
    d = neigh_hbm.shape[2]

    def copies(blk, slot):
        return [
            pltpu.make_async_copy(
                neigh_hbm.at[pl.ds(blk * _BN + s * _RS, _RS)],
                nbuf.at[slot, s],
                sems.at[slot, s],
            )
            for s in range(_NSUB)
        ]

    @pl.when(i == 0)
    def _():
        for c in copies(0, 0):
            c.start()

    @pl.when(i + 1 < ngrid)
    def _():
        for c in copies(i + 1, (i + 1) % 2):
            c.start()

    for c in copies(i, i % 2):
        c.wait()

    blk = nbuf[i % 2]
    s = jnp.sum(blk, axis=2).reshape(_BN, d) + prev_ref[...]
    means = s * (1.0 / (k + 1))
    acc = jnp.dot(means, w_ref[...], preferred_element_type=jnp.float32)
    out_ref[...] = jnp.maximum(acc, 0.0)


def _tail_body(_, means_ref, w_ref, out_ref):
    acc = jnp.dot(means_ref[...], w_ref[...], preferred_element_type=jnp.float32)
    out_ref[...] = jnp.maximum(acc, 0.0)


def _sc_means(prev_hidden, neigh_hidden, nt):
    n, d = prev_hidden.shape
    k = neigh_hidden.shape[1]
    nr = n - nt
    nchunk_total = nr // _CH
    base_q, rem = divmod(nchunk_total, _NW)
    inv = 1.0 / (k + 1)
    mesh = plsc.VectorSubcoreMesh(core_axis_name="c", subcore_axis_name="s")

    cpw = base_q + (1 if rem else 0)

    @functools.partial(
        pl.kernel,
        out_type=jax.ShapeDtypeStruct((nr, d), jnp.float32),
        mesh=mesh,
        scratch_types=[
            pltpu.VMEM((2, _CH, k, d), jnp.float32),
            pltpu.VMEM((2, _CH, d), jnp.float32),
            pltpu.VMEM((_CH, d), jnp.float32),
            pltpu.SemaphoreType.DMA,
            pltpu.SemaphoreType.DMA,
        ],
    )
    def sc_kernel(prev_hbm, neigh_hbm, means_hbm, neigh_v, prev_v, out_v,
                  sem0, sem1):
        wid = lax.axis_index("s") * _NC + lax.axis_index("c")
        c_start = wid * base_q + jnp.minimum(wid, rem)
        c_last = c_start + base_q + jnp.where(wid < rem, 1, 0) - 1
        sems = (sem0, sem1)
        ngrp = d // 16

        def start_fetch(i):
            ci = jnp.minimum(c_start + i, c_last)
            b = i % 2
            r0 = ci * _CH
            dn = pltpu.async_copy(
                neigh_hbm.at[pl.ds(nt + r0, _CH)], neigh_v.at[b], sems[b])
            dp = pltpu.async_copy(
                prev_hbm.at[pl.ds(nt + r0, _CH)], prev_v.at[b], sems[b])
            return (dn, dp)

        pending = start_fetch(0)
        for i in range(cpw):
            b = i % 2
            nxt = start_fetch(i + 1) if i + 1 < cpw else None
            pending[0].wait()
            pending[1].wait()
            pending = nxt

            @plsc.parallel_loop(0, _CH * ngrp, unroll=2)
            def _(idx):
                sh = ngrp.bit_length() - 1
                r = lax.shift_right_logical(idx, sh)
                col = pl.multiple_of(
                    lax.shift_left(jnp.bitwise_and(idx, ngrp - 1), 4), 16)
                acc = prev_v[b, r, pl.ds(col, 16)]
                for kk in range(k):
                    acc = acc + neigh_v[b, r, kk, pl.ds(col, 16)]
                out_v[r, pl.ds(col, 16)] = acc * inv

            ci = jnp.minimum(c_start + i, c_last)
            pltpu.sync_copy(out_v, means_hbm.at[pl.ds(ci * _CH, _CH)])

    return sc_kernel(prev_hidden, neigh_hidden)


def kernel(prev_hidden, neigh_hidden, W):
    n, d = prev_hidden.shape
    k = neigh_hidden.shape[1]
    f = W.shape[1]
    nt = _NT
    nr = n - nt

    if nr == 0:
        return pl.pallas_call(
            _head_body,
            grid=(nt // _BN,),
            in_specs=[
                pl.BlockSpec(memory_space=pltpu.MemorySpace.HBM),
                pl.BlockSpec((_BN, d), lambda i: (i, 0)),
                pl.BlockSpec((d, f), lambda i: (0, 0)),
            ],
            out_specs=pl.BlockSpec((_BN, f), lambda i: (i, 0)),
            out_shape=jax.ShapeDtypeStruct((n, f), jnp.float32),
            scratch_shapes=[
                pltpu.VMEM((2, _NSUB, _RS, k, d), jnp.float32),
                pltpu.SemaphoreType.DMA((2, _NSUB)),
            ],
        )(neigh_hidden, prev_hidden, W)

    means = _sc_means(prev_hidden, neigh_hidden, nt)

    out_head = pl.pallas_call(
        _head_body,
        grid=(nt // _BN,),
        in_specs=[
            pl.BlockSpec(memory_space=pltpu.MemorySpace.HBM),
            pl.BlockSpec((_BN, d), lambda i: (i, 0)),
            pl.BlockSpec((d, f), lambda i: (0, 0)),
        ],
        out_specs=pl.BlockSpec((_BN, f), lambda i: (i, 0)),
        out_shape=jax.ShapeDtypeStruct((n, f), jnp.float32),
        scratch_shapes=[
            pltpu.VMEM((2, _NSUB, _RS, k, d), jnp.float32),
            pltpu.SemaphoreType.DMA((2, _NSUB)),
        ],
    )(neigh_hidden, prev_hidden, W)

    off = nt // _BNC
    out = pl.pallas_call(
        _tail_body,
        grid=(nr // _BNC,),
        in_specs=[
            pl.BlockSpec(memory_space=pltpu.MemorySpace.HBM),
            pl.BlockSpec((_BNC, d), lambda i: (i, 0)),
            pl.BlockSpec((d, f), lambda i: (0, 0)),
        ],
        out_specs=pl.BlockSpec((_BNC, f), lambda i: (i + off, 0)),
        out_shape=jax.ShapeDtypeStruct((n, f), jnp.float32),
        input_output_aliases={0: 0},
    )(out_head, means, W)
    return out
